# baseline (speedup 1.0000x reference)
import functools

import jax
import jax.numpy as jnp
from jax import lax
from jax.experimental import pallas as pl
from jax.experimental.pallas import tpu as pltpu
from jax.experimental.pallas import tpu_sc as plsc

NUM_USERS = 30000
NUM_ITEMS = 20000
NUM_AUTHORS = 10000
DIM = 64
HALF = 32
N_LAYERS = 3

NSC = 2
NSUB = 16
SUB = 128
NK = 3
CH = SUB * NK
EALIGN = CH * NSUB
ZR = 200


def _pad_edges(row, col, val):
  e = row.shape[0]
  ep = -(-e // EALIGN) * EALIGN
  pad = ep - e
  if pad == 0:
    return row, col, val
  return (jnp.pad(row, (0, pad)), jnp.pad(col, (0, pad)),
          jnp.pad(val, (0, pad)))


GN = 20480
GCH = 128


def _make_spmm(E, n_src, n_dst, n_layers, gather_src=None):
  E = -(-E // EALIGN) * EALIGN
  assert E % EALIGN == 0
  nchunks = E // CH // NSUB
  assert nchunks >= 2
  eper = E // NSUB
  assert n_dst % ZR == 0
  ntiles = n_dst // ZR

  mesh = plsc.VectorSubcoreMesh(
      core_axis_name="c", subcore_axis_name="s",
      num_cores=NSC, num_subcores=NSUB)

  out_type = jax.ShapeDtypeStruct((n_layers, NSC, n_dst, HALF), jnp.float32)
  if gather_src is not None:
    out_type = (out_type,
                jax.ShapeDtypeStruct((NSC, GN, HALF), jnp.float32))
  scratch = [
      pltpu.VMEM((2 * NK * SUB,), jnp.int32),
      pltpu.VMEM((2 * NK * SUB,), jnp.int32),
      pltpu.VMEM((NK, SUB), jnp.int32),
      pltpu.VMEM((2 * CH,), jnp.float32),
      pltpu.VMEM((2 * CH, HALF), jnp.float32),
      pltpu.SemaphoreType.DMA,
      pltpu.SemaphoreType.DMA,
      pltpu.SemaphoreType.DMA,
      pltpu.VMEM_SHARED((n_dst, HALF), jnp.float32),
  ]

  @functools.partial(pl.kernel, out_type=out_type, mesh=mesh,
                     scratch_types=scratch,
                     compiler_params=pltpu.CompilerParams(
                         use_tc_tiling_on_sc=False))
  def spmm_kernel(row_h, col_h, val_h, *refs):
    if gather_src is None:
      gidx_h = out2_h = None
      (x_h, out_h, colb, rowb, rowb2, valb, rows,
       sem_lin, sem_g, sem_s, acc) = refs
    else:
      (gidx_h, x_h, out_h, out2_h, colb, rowb, rowb2, valb, rows,
       sem_lin, sem_g, sem_s, acc) = refs
    c = lax.axis_index("c")
    s = lax.axis_index("s")
    zeros16 = jnp.zeros((16,), jnp.float32)

    my_t = (ntiles - s + (NSUB - 1)) // NSUB

    for l in range(n_layers):
      def zfill(r, carry):
        rows[r, pl.ds(0, 16)] = zeros16
        rows[r, pl.ds(16, 16)] = zeros16
        return carry

      lax.fori_loop(0, ZR, zfill, 0)

      def zero_body(z, carry):
        pltpu.sync_copy(rows.at[pl.ds(0, ZR)],
                        acc.at[pl.ds((s + z * NSUB) * ZR, ZR)])
        return carry

      lax.fori_loop(0, my_t, zero_body, 0)
      plsc.subcore_barrier()

      src = x_h if l == 0 else out_h.at[l - 1]
      sbase = s * eper

      def fire_idx(chunk, slot):
        base = sbase + chunk * CH
        io = slot * (NK * SUB)
        for k in range(NK):
          pltpu.async_copy(col_h.at[pl.ds(base + k * SUB, SUB)],
                           colb.at[pl.ds(io + k * SUB, SUB)], sem_lin)
          pltpu.async_copy(row_h.at[pl.ds(base + k * SUB, SUB)],
                           rowb.at[pl.ds(io + k * SUB, SUB)], sem_lin)
        pltpu.async_copy(val_h.at[pl.ds(base, CH)],
                         valb.at[pl.ds(slot * CH, CH)], sem_lin)

      def drain_idx(slot):
        io = slot * (NK * SUB)
        for k in range(NK):
          pltpu.make_async_copy(col_h.at[pl.ds(sbase, SUB)],
                                colb.at[pl.ds(io + k * SUB, SUB)],
                                sem_lin).wait()
          pltpu.make_async_copy(row_h.at[pl.ds(sbase, SUB)],
                                rowb.at[pl.ds(io + k * SUB, SUB)],
                                sem_lin).wait()
        pltpu.make_async_copy(val_h.at[pl.ds(sbase, CH)],
                              valb.at[pl.ds(slot * CH, CH)], sem_lin).wait()

      def fire_gathers(slot):
        io = slot * (NK * SUB)
        for k in range(NK):
          pltpu.async_copy(src.at[c].at[colb.at[pl.ds(io + k * SUB, SUB)]],
                           rows.at[pl.ds(io + k * SUB, SUB)], sem_g)

      def drain_scatters(slot):
        io = slot * (NK * SUB)
        for k in range(NK):
          pltpu.make_async_copy(rows.at[pl.ds(io + k * SUB, SUB)],
                                acc.at[rowb2.at[k]], sem_s).wait()

      fire_idx(0, jnp.int32(0))
      drain_idx(jnp.int32(0))
      fire_idx(1, jnp.int32(1))
      fire_gathers(jnp.int32(0))

      def chunk_body(i, carry):
        p = lax.rem(i, 2)
        q = 1 - p
        @pl.when(i > 0)
        def _():
          drain_scatters(q)
        io = p * (NK * SUB)
        for k in range(NK):
          for g in range(SUB // 16):
            rowb2[k, pl.ds(g * 16, 16)] = (
                rowb[pl.ds(io + k * SUB + g * 16, 16)])
        @pl.when(i < nchunks - 1)
        def _():
          drain_idx(q)
        for k in range(NK):
          pltpu.make_async_copy(src.at[c].at[colb.at[pl.ds(io, SUB)]],
                                rows.at[pl.ds(io + k * SUB, SUB)],
                                sem_g).wait()
          for g in range(SUB // 16):
            off = io + k * SUB + g * 16
            v16 = valb[pl.ds(p * CH + k * SUB + g * 16, 16)]
            for t in range(16):
              v = v16[t]
              rows[off + t, pl.ds(0, 16)] = rows[off + t, pl.ds(0, 16)] * v
              rows[off + t, pl.ds(16, 16)] = rows[off + t, pl.ds(16, 16)] * v
          pltpu.async_copy(rows.at[pl.ds(io + k * SUB, SUB)],
                           acc.at[rowb2.at[k]], sem_s, add=True)
        @pl.when(i < nchunks - 2)
        def _():
          fire_idx(i + 2, p)
        @pl.when(i < nchunks - 1)
        def _():
          fire_gathers(q)
        return carry

      lax.fori_loop(0, nchunks, chunk_body, 0)
      drain_scatters(jnp.int32((nchunks - 1) % 2))
      plsc.subcore_barrier()

      def wb_body(z, carry):
        off = (s + z * NSUB) * ZR
        pltpu.sync_copy(acc.at[pl.ds(off, ZR)], rows.at[pl.ds(0, ZR)])
        pltpu.sync_copy(rows.at[pl.ds(0, ZR)], out_h.at[l, c, pl.ds(off, ZR)])
        return carry

      lax.fori_loop(0, my_t, wb_body, 0)
      plsc.subcore_barrier()

    if gather_src is not None:
      gsrc = x_h if gather_src == "x" else out_h.at[n_layers - 1]
      my_g = (GN // GCH - s + (NSUB - 1)) // NSUB

      def gather_body(i, carry):
        off = (s + i * NSUB) * GCH
        pltpu.sync_copy(gidx_h.at[pl.ds(off, GCH)], colb.at[pl.ds(0, GCH)])
        pltpu.async_copy(gsrc.at[c].at[colb.at[pl.ds(0, GCH)]],
                         rows.at[pl.ds(0, GCH)], sem_g).wait()
        pltpu.sync_copy(rows.at[pl.ds(0, GCH)], out2_h.at[c, pl.ds(off, GCH)])
        return carry

      lax.fori_loop(0, my_g, gather_body, 0)

  return spmm_kernel


_spmm_g0 = _make_spmm(800000, NUM_USERS + NUM_ITEMS, NUM_USERS + NUM_ITEMS,
                      N_LAYERS)
_spmm_g1 = _make_spmm(640000, NUM_USERS + NUM_AUTHORS,
                      NUM_USERS + NUM_AUTHORS, N_LAYERS)
_spmm_g3 = _make_spmm(160000, NUM_ITEMS, NUM_AUTHORS, 1, gather_src="out")
_spmm_g5 = _make_spmm(320000, NUM_ITEMS, NUM_ITEMS, 1)
_spmm_g4 = _make_spmm(320000, NUM_AUTHORS, NUM_ITEMS, 1, gather_src="x")


ITEM_PAD = 20480
IB = 512


def _score_body(u1_ref, u2_ref, ai_ref, ni_ref, aa_ref, an_ref, q_ref, o_ref):
  u1 = u1_ref[...]
  u2 = u2_ref[...]
  ai = ai_ref[...]
  ni = ni_ref[...]
  aa = aa_ref[...]
  an = an_ref[...]
  dn = (((1,), (0,)), ((), ()))
  ui = jax.nn.sigmoid(
      lax.dot_general(u1, ai, dn, preferred_element_type=jnp.float32)
      + lax.dot_general(u2, ni, dn, preferred_element_type=jnp.float32))
  ua = jax.nn.sigmoid(
      lax.dot_general(u1, aa, dn, preferred_element_type=jnp.float32)
      + lax.dot_general(u2, an, dn, preferred_element_type=jnp.float32))
  itf = (ai + ni) * 0.5
  auf = (aa + an) * 0.5
  dq = (((0,), (0,)), ((), ()))
  cq = lax.dot_general(q_ref[...], itf, dq, preferred_element_type=jnp.float32)
  w = jax.nn.sigmoid(jnp.sum(cq * auf, axis=0))
  o_ref[...] = w[None, :] * ui + (1.0 - w)[None, :] * ua


def _score(u_atom, u_non, ai_t, ni_t, aa_t, an_t, q):
  nu = u_atom.shape[0]
  grid = (ITEM_PAD // IB,)
  ublock = pl.BlockSpec((nu, DIM), lambda i: (0, 0))
  iblock = pl.BlockSpec((DIM, IB), lambda i: (0, i))
  return pl.pallas_call(
      _score_body,
      grid=grid,
      in_specs=[ublock, ublock, iblock, iblock, iblock, iblock,
                pl.BlockSpec((DIM, DIM), lambda i: (0, 0))],
      out_specs=pl.BlockSpec((nu, IB), lambda i: (0, i)),
      out_shape=jax.ShapeDtypeStruct((nu, ITEM_PAD), jnp.float32),
  )(u_atom, u_non, ai_t, ni_t, aa_t, an_t, q)


def _halves(x):
  return jnp.stack([x[:, :HALF], x[:, HALF:]], axis=0)


def _unhalve(xh):
  return jnp.concatenate([xh[0], xh[1]], axis=1)


def _normalize(x, eps=1e-12):
  n = jnp.sqrt(jnp.sum(x * x, axis=1, keepdims=True))
  return x / jnp.maximum(n, eps)


def _pad_items_t(x):
  return jnp.pad(x.T, ((0, 0), (0, ITEM_PAD - NUM_ITEMS)))


def kernel(users, g0_idx, g0_val, g1_idx, g1_val, g3_row, g3_col, g3_val,
           g5_idx, g5_val, g4_row, g4_col, g4_val, author_list,
           emb_user, emb_item, emb_author, q):
  i32 = jnp.int32
  alist = jnp.pad(author_list.astype(i32), (0, GN - NUM_ITEMS))
  x0h = _halves(jnp.concatenate([emb_user, emb_item], axis=0))
  f0 = _spmm_g0(*_pad_edges(g0_idx[0].astype(i32), g0_idx[1].astype(i32),
                            g0_val), x0h)
  items0 = (x0h[:, NUM_USERS:] + f0[0][:, NUM_USERS:] + f0[1][:, NUM_USERS:]
            + f0[2][:, NUM_USERS:]) * 0.25
  u_atom = _unhalve((x0h[:, users] + f0[0][:, users] + f0[1][:, users]
                     + f0[2][:, users]) * 0.25)
  atom_items = _unhalve(items0)

  s3, a_atom_h = _spmm_g3(*_pad_edges(g3_row.astype(i32), g3_col.astype(i32),
                                      g3_val), alist, items0)
  a_atom = _normalize(_unhalve(a_atom_h)[:NUM_ITEMS])
  s5 = _spmm_g5(*_pad_edges(g5_idx[0].astype(i32), g5_idx[1].astype(i32),
                            g5_val), items0)
  atom_items = 0.5 * _normalize(_unhalve(s5[0])) + 0.5 * atom_items

  x1h = _halves(jnp.concatenate([emb_user, emb_author], axis=0))
  f1 = _spmm_g1(*_pad_edges(g1_idx[0].astype(i32), g1_idx[1].astype(i32),
                            g1_val), x1h)
  auth1 = (x1h[:, NUM_USERS:] + f1[0][:, NUM_USERS:] + f1[1][:, NUM_USERS:]
           + f1[2][:, NUM_USERS:]) * 0.25
  u_non = _unhalve((x1h[:, users] + f1[0][:, users] + f1[1][:, users]
                    + f1[2][:, users]) * 0.25)

  s4, a_non_h = _spmm_g4(*_pad_edges(g4_row.astype(i32), g4_col.astype(i32),
                                     g4_val), alist, auth1)
  non_items = _normalize(_unhalve(s4[0]))
  a_non = _unhalve(a_non_h)[:NUM_ITEMS]

  out = _score(u_atom, u_non, _pad_items_t(atom_items), _pad_items_t(non_items),
               _pad_items_t(a_atom), _pad_items_t(a_non), q)
  return out[:, :NUM_ITEMS]


# speedup vs baseline: 1.2660x; 1.2660x over previous
import functools

import jax
import jax.numpy as jnp
from jax import lax
from jax.experimental import pallas as pl
from jax.experimental.pallas import tpu as pltpu
from jax.experimental.pallas import tpu_sc as plsc

NUM_USERS = 30000
NUM_ITEMS = 20000
NUM_AUTHORS = 10000
DIM = 64
HALF = 32
N_LAYERS = 3

NSC = 2
NSUB = 16
SUB = 128
NK = 3
CH = SUB * NK
EALIGN = CH * NSUB
ZR = 200


def _pad_edges(row, col, val):
  e = row.shape[0]
  ep = -(-e // EALIGN) * EALIGN
  pad = ep - e
  if pad == 0:
    return row, col, val
  return (jnp.pad(row, (0, pad)), jnp.pad(col, (0, pad)),
          jnp.pad(val, (0, pad)))


GN = 20480
GCH = 128


def _make_spmm(E, n_src, n_dst, n_layers, gather_src=None):
  E = -(-E // EALIGN) * EALIGN
  assert E % EALIGN == 0
  nchunks = E // CH // NSUB
  assert nchunks >= 2
  eper = E // NSUB
  assert n_dst % ZR == 0
  ntiles = n_dst // ZR

  mesh = plsc.VectorSubcoreMesh(
      core_axis_name="c", subcore_axis_name="s",
      num_cores=NSC, num_subcores=NSUB)

  out_type = jax.ShapeDtypeStruct((n_layers, NSC, n_dst, HALF), jnp.float32)
  if gather_src is not None:
    out_type = (out_type,
                jax.ShapeDtypeStruct((NSC, GN, HALF), jnp.float32))
  scratch = [
      pltpu.VMEM((2 * NK * SUB,), jnp.int32),
      pltpu.VMEM((2 * NK * SUB,), jnp.int32),
      pltpu.VMEM((NK, SUB), jnp.int32),
      pltpu.VMEM((2 * CH,), jnp.float32),
      pltpu.VMEM((2 * CH, HALF), jnp.float32),
      pltpu.SemaphoreType.DMA,
      pltpu.SemaphoreType.DMA,
      pltpu.SemaphoreType.DMA,
      pltpu.VMEM_SHARED((n_dst, HALF), jnp.float32),
  ]

  @functools.partial(pl.kernel, out_type=out_type, mesh=mesh,
                     scratch_types=scratch,
                     compiler_params=pltpu.CompilerParams(
                         use_tc_tiling_on_sc=False))
  def spmm_kernel(row_h, col_h, val_h, *refs):
    if gather_src is None:
      gidx_h = out2_h = None
      (x_h, out_h, colb, rowb, rowb2, valb, rows,
       sem_lin, sem_g, sem_s, acc) = refs
    else:
      (gidx_h, x_h, out_h, out2_h, colb, rowb, rowb2, valb, rows,
       sem_lin, sem_g, sem_s, acc) = refs
    c = lax.axis_index("c")
    s = lax.axis_index("s")
    zeros16 = jnp.zeros((16,), jnp.float32)

    my_t = (ntiles - s + (NSUB - 1)) // NSUB

    for l in range(n_layers):
      def zfill(r, carry):
        rows[r, pl.ds(0, 16)] = zeros16
        rows[r, pl.ds(16, 16)] = zeros16
        return carry

      lax.fori_loop(0, ZR, zfill, 0)

      def zero_body(z, carry):
        pltpu.sync_copy(rows.at[pl.ds(0, ZR)],
                        acc.at[pl.ds((s + z * NSUB) * ZR, ZR)])
        return carry

      lax.fori_loop(0, my_t, zero_body, 0)
      plsc.subcore_barrier()

      src = x_h if l == 0 else out_h.at[l - 1]
      sbase = s * eper

      def fire_idx(chunk, slot):
        base = sbase + chunk * CH
        io = slot * (NK * SUB)
        for k in range(NK):
          pltpu.async_copy(col_h.at[pl.ds(base + k * SUB, SUB)],
                           colb.at[pl.ds(io + k * SUB, SUB)], sem_lin)
          pltpu.async_copy(row_h.at[pl.ds(base + k * SUB, SUB)],
                           rowb.at[pl.ds(io + k * SUB, SUB)], sem_lin)
        pltpu.async_copy(val_h.at[pl.ds(base, CH)],
                         valb.at[pl.ds(slot * CH, CH)], sem_lin)

      def drain_idx(slot):
        io = slot * (NK * SUB)
        for k in range(NK):
          pltpu.make_async_copy(col_h.at[pl.ds(sbase, SUB)],
                                colb.at[pl.ds(io + k * SUB, SUB)],
                                sem_lin).wait()
          pltpu.make_async_copy(row_h.at[pl.ds(sbase, SUB)],
                                rowb.at[pl.ds(io + k * SUB, SUB)],
                                sem_lin).wait()
        pltpu.make_async_copy(val_h.at[pl.ds(sbase, CH)],
                              valb.at[pl.ds(slot * CH, CH)], sem_lin).wait()

      def fire_gathers(slot):
        io = slot * (NK * SUB)
        for k in range(NK):
          pltpu.async_copy(src.at[c].at[colb.at[pl.ds(io + k * SUB, SUB)]],
                           rows.at[pl.ds(io + k * SUB, SUB)], sem_g)

      def drain_scatters(slot):
        io = slot * (NK * SUB)
        for k in range(NK):
          pltpu.make_async_copy(rows.at[pl.ds(io + k * SUB, SUB)],
                                acc.at[rowb2.at[k]], sem_s).wait()

      fire_idx(0, jnp.int32(0))
      drain_idx(jnp.int32(0))
      fire_idx(1, jnp.int32(1))
      fire_gathers(jnp.int32(0))

      def chunk_body(i, carry):
        p = lax.rem(i, 2)
        q = 1 - p
        @pl.when(i > 0)
        def _():
          drain_scatters(q)
        io = p * (NK * SUB)
        for k in range(NK):
          for g in range(SUB // 16):
            rowb2[k, pl.ds(g * 16, 16)] = (
                rowb[pl.ds(io + k * SUB + g * 16, 16)])
        @pl.when(i < nchunks - 1)
        def _():
          drain_idx(q)
        for k in range(NK):
          pltpu.make_async_copy(src.at[c].at[colb.at[pl.ds(io, SUB)]],
                                rows.at[pl.ds(io + k * SUB, SUB)],
                                sem_g).wait()
          for g in range(SUB // 16):
            off = io + k * SUB + g * 16
            v16 = valb[pl.ds(p * CH + k * SUB + g * 16, 16)]
            for t in range(16):
              v = v16[t]
              rows[off + t, pl.ds(0, 16)] = rows[off + t, pl.ds(0, 16)] * v
              rows[off + t, pl.ds(16, 16)] = rows[off + t, pl.ds(16, 16)] * v
          pltpu.async_copy(rows.at[pl.ds(io + k * SUB, SUB)],
                           acc.at[rowb2.at[k]], sem_s, add=True)
        @pl.when(i < nchunks - 2)
        def _():
          fire_idx(i + 2, p)
        @pl.when(i < nchunks - 1)
        def _():
          fire_gathers(q)
        return carry

      lax.fori_loop(0, nchunks, chunk_body, 0)
      drain_scatters(jnp.int32((nchunks - 1) % 2))
      plsc.subcore_barrier()

      def wb_body(z, carry):
        off = (s + z * NSUB) * ZR
        pltpu.sync_copy(acc.at[pl.ds(off, ZR)], rows.at[pl.ds(0, ZR)])
        pltpu.sync_copy(rows.at[pl.ds(0, ZR)], out_h.at[l, c, pl.ds(off, ZR)])
        return carry

      lax.fori_loop(0, my_t, wb_body, 0)
      plsc.subcore_barrier()

    if gather_src is not None:
      gsrc = x_h if gather_src == "x" else out_h.at[n_layers - 1]
      my_g = (GN // GCH - s + (NSUB - 1)) // NSUB

      def gather_body(i, carry):
        off = (s + i * NSUB) * GCH
        pltpu.sync_copy(gidx_h.at[pl.ds(off, GCH)], colb.at[pl.ds(0, GCH)])
        pltpu.async_copy(gsrc.at[c].at[colb.at[pl.ds(0, GCH)]],
                         rows.at[pl.ds(0, GCH)], sem_g).wait()
        pltpu.sync_copy(rows.at[pl.ds(0, GCH)], out2_h.at[c, pl.ds(off, GCH)])
        return carry

      lax.fori_loop(0, my_g, gather_body, 0)

  return spmm_kernel


_spmm_g0 = _make_spmm(800000, NUM_USERS + NUM_ITEMS, NUM_USERS + NUM_ITEMS,
                      N_LAYERS)
_spmm_g1 = _make_spmm(640000, NUM_USERS + NUM_AUTHORS,
                      NUM_USERS + NUM_AUTHORS, N_LAYERS)
_spmm_g3 = _make_spmm(160000, NUM_ITEMS, NUM_AUTHORS, 1)
_spmm_g5 = _make_spmm(320000, NUM_ITEMS, NUM_ITEMS, 1)
_spmm_g4 = _make_spmm(320000, NUM_AUTHORS, NUM_ITEMS, 1)


ITEM_PAD = 20480
IB = 512


def _score_body(u1_ref, u2_ref, ai_ref, ni_ref, aa_ref, an_ref, q_ref, o_ref):
  u1 = u1_ref[...]
  u2 = u2_ref[...]
  ai = ai_ref[...]
  ni = ni_ref[...]
  aa = aa_ref[...]
  an = an_ref[...]
  dn = (((1,), (0,)), ((), ()))
  ui = jax.nn.sigmoid(
      lax.dot_general(u1, ai, dn, preferred_element_type=jnp.float32)
      + lax.dot_general(u2, ni, dn, preferred_element_type=jnp.float32))
  ua = jax.nn.sigmoid(
      lax.dot_general(u1, aa, dn, preferred_element_type=jnp.float32)
      + lax.dot_general(u2, an, dn, preferred_element_type=jnp.float32))
  itf = (ai + ni) * 0.5
  auf = (aa + an) * 0.5
  dq = (((0,), (0,)), ((), ()))
  cq = lax.dot_general(q_ref[...], itf, dq, preferred_element_type=jnp.float32)
  w = jax.nn.sigmoid(jnp.sum(cq * auf, axis=0))
  o_ref[...] = w[None, :] * ui + (1.0 - w)[None, :] * ua


def _score(u_atom, u_non, ai_t, ni_t, aa_t, an_t, q):
  nu = u_atom.shape[0]
  grid = (ITEM_PAD // IB,)
  ublock = pl.BlockSpec((nu, DIM), lambda i: (0, 0))
  iblock = pl.BlockSpec((DIM, IB), lambda i: (0, i))
  return pl.pallas_call(
      _score_body,
      grid=grid,
      in_specs=[ublock, ublock, iblock, iblock, iblock, iblock,
                pl.BlockSpec((DIM, DIM), lambda i: (0, 0))],
      out_specs=pl.BlockSpec((nu, IB), lambda i: (0, i)),
      out_shape=jax.ShapeDtypeStruct((nu, ITEM_PAD), jnp.float32),
  )(u_atom, u_non, ai_t, ni_t, aa_t, an_t, q)


def _halves(x):
  return jnp.stack([x[:, :HALF], x[:, HALF:]], axis=0)


def _unhalve(xh):
  return jnp.concatenate([xh[0], xh[1]], axis=1)


def _normalize(x, eps=1e-12):
  n = jnp.sqrt(jnp.sum(x * x, axis=1, keepdims=True))
  return x / jnp.maximum(n, eps)


def _pad_items_t(x):
  return jnp.pad(x.T, ((0, 0), (0, ITEM_PAD - NUM_ITEMS)))


def kernel(users, g0_idx, g0_val, g1_idx, g1_val, g3_row, g3_col, g3_val,
           g5_idx, g5_val, g4_row, g4_col, g4_val, author_list,
           emb_user, emb_item, emb_author, q):
  i32 = jnp.int32
  x0h = _halves(jnp.concatenate([emb_user, emb_item], axis=0))
  f0 = _spmm_g0(*_pad_edges(g0_idx[0].astype(i32), g0_idx[1].astype(i32),
                            g0_val), x0h)
  light0 = (x0h + f0[0] + f0[1] + f0[2]) * 0.25
  atom = _unhalve(light0)
  atom_users = atom[:NUM_USERS]
  atom_items = atom[NUM_USERS:]

  aih = _halves(atom_items)
  s3 = _spmm_g3(*_pad_edges(g3_row.astype(i32), g3_col.astype(i32), g3_val),
                aih)
  atom_authors = _normalize(_unhalve(s3[0]))
  s5 = _spmm_g5(*_pad_edges(g5_idx[0].astype(i32), g5_idx[1].astype(i32),
                            g5_val), aih)
  atom_items = 0.5 * _normalize(_unhalve(s5[0])) + 0.5 * atom_items

  x1h = _halves(jnp.concatenate([emb_user, emb_author], axis=0))
  f1 = _spmm_g1(*_pad_edges(g1_idx[0].astype(i32), g1_idx[1].astype(i32),
                            g1_val), x1h)
  light1 = (x1h + f1[0] + f1[1] + f1[2]) * 0.25
  non = _unhalve(light1)
  non_users = non[:NUM_USERS]
  non_authors = non[NUM_USERS:]

  nah = _halves(non_authors)
  s4 = _spmm_g4(*_pad_edges(g4_row.astype(i32), g4_col.astype(i32), g4_val),
                nah)
  non_items = _normalize(_unhalve(s4[0]))

  u_atom = atom_users[users]
  u_non = non_users[users]
  a_atom = atom_authors[author_list]
  a_non = non_authors[author_list]
  out = _score(u_atom, u_non, _pad_items_t(atom_items), _pad_items_t(non_items),
               _pad_items_t(a_atom), _pad_items_t(a_non), q)
  return out[:, :NUM_ITEMS]
